# Initial kernel scaffold; baseline (speedup 1.0000x reference)
import jax
import jax.numpy as jnp
from jax import lax
from jax.experimental import pallas as pl
from jax.experimental.pallas import tpu as pltpu
from jax.experimental.pallas import tpu_sc as plsc

N = 10000
E = 320000
D_IN = 128
D_OUT = 16

NC = 2
NS = 16
L = 16

E_VALID = E + N
CHUNK = 128
NW = NC * NS
K_CHUNKS = -(-E_VALID // (CHUNK * NW))
E_TILE = K_CHUNKS * CHUNK
E_PAD = E_TILE * NW
N_PAD = 10240
ROWS_T = N_PAD // NS
G_TILE = N_PAD // NW
G_CHUNK = 64
G_STEPS = G_TILE // G_CHUNK


def _leaky(x):
    return jnp.where(x >= 0, x, 0.2 * x)


def _dense_body(x1, x2, w1, w2, s1, d1, s2, d2,
                xp1_o, xp2_o, as1_o, ad1_o, m1_o, as2_o, ad2_o, m2_o):
    xp1 = jnp.dot(x1[...], w1[...], preferred_element_type=jnp.float32)
    xp2 = jnp.dot(x2[...], w2[...], preferred_element_type=jnp.float32)
    xp1_o[...] = xp1
    xp2_o[...] = xp2
    as1 = jnp.sum(xp1 * s1[...], axis=1, keepdims=True)
    ad1 = jnp.sum(xp1 * d1[...], axis=1, keepdims=True)
    as2 = jnp.sum(xp2 * s2[...], axis=1, keepdims=True)
    ad2 = jnp.sum(xp2 * d2[...], axis=1, keepdims=True)
    as1_o[...] = as1
    ad1_o[...] = ad1
    as2_o[...] = as2
    ad2_o[...] = ad2
    m1_o[...] = _leaky(jnp.max(as1) + ad1)
    m2_o[...] = _leaky(jnp.max(as2) + ad2)


def _dense_call(x1, x2, w1, w2, s1, d1, s2, d2):
    f32 = jnp.float32
    out_shape = [jax.ShapeDtypeStruct((N, D_OUT), f32)] * 2 + \
                [jax.ShapeDtypeStruct((N, 1), f32)] * 6
    return pl.pallas_call(_dense_body, out_shape=out_shape)(
        x1, x2, w1, w2, s1.reshape(1, D_OUT), d1.reshape(1, D_OUT),
        s2.reshape(1, D_OUT), d2.reshape(1, D_OUT))


def _edges_one_gat(c, s, src_h, dst_h, as_h, ad_h, m_h, xp_h, ou_h, dn_h,
                   as_t, ad_t, m_t, den_t, zrow, src_c, dst_c, e_c, rows,
                   acc, sem):
    iota16 = lax.iota(jnp.int32, L)

    pltpu.sync_copy(as_h, as_t)
    pltpu.sync_copy(ad_h, ad_t)
    pltpu.sync_copy(m_h, m_t)

    def _zero_den(j, _):
        den_t[pl.ds(j * L, L)] = jnp.zeros((L,), jnp.float32)
        return 0
    lax.fori_loop(0, N_PAD // L, _zero_den, 0)
    pltpu.sync_copy(zrow, acc.at[pl.ds(s * ROWS_T, ROWS_T)])
    plsc.subcore_barrier()

    base0 = c * (NS * E_TILE) + s * E_TILE

    def _chunk(k, _):
        base = base0 + k * CHUNK
        pltpu.sync_copy(src_h.at[pl.ds(base, CHUNK)], src_c)
        pltpu.sync_copy(dst_h.at[pl.ds(base, CHUNK)], dst_c)
        row_cp = pltpu.async_copy(xp_h.at[src_c], rows, sem)

        def _evec(j, _):
            si = src_c[pl.ds(j * L, L)]
            di = dst_c[pl.ds(j * L, L)]
            a = plsc.load_gather(as_t, [si]) + plsc.load_gather(ad_t, [di])
            a = jnp.where(a >= 0, a, 0.2 * a)
            e = jnp.exp(a - plsc.load_gather(m_t, [di]))
            gid = base + j * L + iota16
            e = jnp.where(gid < E_VALID, e, 0.0)
            e_c[pl.ds(j * L, L)] = e
            plsc.addupdate_scatter(den_t, [di], e)
            return 0
        lax.fori_loop(0, CHUNK // L, _evec, 0, unroll=True)

        row_cp.wait()

        def _scale(j, _):
            rows[j, :] = rows[j, :] * e_c[j]
            return 0
        lax.fori_loop(0, CHUNK, _scale, 0, unroll=8)

        pltpu.sync_copy(rows, acc.at[dst_c], add=True)
        return 0

    lax.fori_loop(0, K_CHUNKS, _chunk, 0)
    plsc.subcore_barrier()

    pltpu.sync_copy(acc.at[pl.ds(s * ROWS_T, ROWS_T)],
                    ou_h.at[c, pl.ds(s * ROWS_T, ROWS_T)])
    pltpu.sync_copy(den_t, dn_h.at[c, s])
    plsc.subcore_barrier()


def _edge_body(src1, dst1, src2, dst2, as1, ad1, m1, as2, ad2, m2, xp1, xp2,
               ou1, dn1, ou2, dn2,
               as_t, ad_t, m_t, den_t, zrow, src_c, dst_c, e_c, rows, acc,
               sem):
    c = lax.axis_index("c")
    s = lax.axis_index("s")

    def _zfill(j, _):
        zrow[j, :] = jnp.zeros((L,), jnp.float32)
        return 0
    lax.fori_loop(0, ROWS_T, _zfill, 0)

    _edges_one_gat(c, s, src1, dst1, as1, ad1, m1, xp1, ou1, dn1,
                   as_t, ad_t, m_t, den_t, zrow, src_c, dst_c, e_c, rows,
                   acc, sem)
    _edges_one_gat(c, s, src2, dst2, as2, ad2, m2, xp2, ou2, dn2,
                   as_t, ad_t, m_t, den_t, zrow, src_c, dst_c, e_c, rows,
                   acc, sem)


def _edge_call(src1, dst1, src2, dst2, as1, ad1, m1, as2, ad2, m2, xp1, xp2):
    f32 = jnp.float32
    mesh = plsc.VectorSubcoreMesh(core_axis_name="c", subcore_axis_name="s")
    out_type = [
        jax.ShapeDtypeStruct((NC, N_PAD, D_OUT), f32),
        jax.ShapeDtypeStruct((NC, NS, N_PAD), f32),
        jax.ShapeDtypeStruct((NC, N_PAD, D_OUT), f32),
        jax.ShapeDtypeStruct((NC, NS, N_PAD), f32),
    ]
    scratch = [
        pltpu.VMEM((N,), f32),
        pltpu.VMEM((N,), f32),
        pltpu.VMEM((N,), f32),
        pltpu.VMEM((N_PAD,), f32),
        pltpu.VMEM((ROWS_T, D_OUT), f32),
        pltpu.VMEM((CHUNK,), jnp.int32),
        pltpu.VMEM((CHUNK,), jnp.int32),
        pltpu.VMEM((CHUNK,), f32),
        pltpu.VMEM((CHUNK, D_OUT), f32),
        pltpu.VMEM_SHARED((N_PAD, D_OUT), f32),
        pltpu.SemaphoreType.DMA,
    ]
    kfn = pl.kernel(_edge_body, out_type=out_type, mesh=mesh,
                    scratch_types=scratch)
    return kfn(src1, dst1, src2, dst2, as1, ad1, m1, as2, ad2, m2, xp1, xp2)


def _norm_body(ou1, dn1, ou2, dn2, b1, b2, x1o, x2o):
    s1 = ou1[0] + ou1[1]
    s2 = ou2[0] + ou2[1]
    d1 = jnp.sum(dn1[...], axis=(0, 1))[:, None]
    d2 = jnp.sum(dn2[...], axis=(0, 1))[:, None]
    x1o[...] = s1 / d1 + b1[...]
    x2o[...] = s2 / d2 + b2[...]


def _norm_call(ou1, dn1, ou2, dn2, b1, b2):
    f32 = jnp.float32
    out_shape = [jax.ShapeDtypeStruct((N_PAD, D_OUT), f32)] * 2
    return pl.pallas_call(_norm_body, out_shape=out_shape)(
        ou1, dn1, ou2, dn2, b1.reshape(1, D_OUT), b2.reshape(1, D_OUT))


def _gather_body(x2o, grp, out, idx_v, rows_v, sem):
    c = lax.axis_index("c")
    s = lax.axis_index("s")
    base = (s * NC + c) * G_TILE

    def _step(k, _):
        off = base + k * G_CHUNK
        pltpu.sync_copy(grp.at[pl.ds(off, G_CHUNK)], idx_v)
        pltpu.async_copy(x2o.at[idx_v], rows_v, sem).wait()
        pltpu.sync_copy(rows_v, out.at[pl.ds(off, G_CHUNK)])
        return 0
    lax.fori_loop(0, G_STEPS, _step, 0)


def _gather_call(x2o, grp):
    mesh = plsc.VectorSubcoreMesh(core_axis_name="c", subcore_axis_name="s")
    out_type = jax.ShapeDtypeStruct((N_PAD, D_OUT), jnp.float32)
    scratch = [
        pltpu.VMEM((G_CHUNK,), jnp.int32),
        pltpu.VMEM((G_CHUNK, D_OUT), jnp.float32),
        pltpu.SemaphoreType.DMA,
    ]
    kfn = pl.kernel(_gather_body, out_type=out_type, mesh=mesh,
                    scratch_types=scratch)
    return kfn(x2o, grp)


def _combine_body(x1o, g, out):
    out[...] = x1o[...] + g[...]


def _combine_call(x1o, g):
    return pl.pallas_call(
        _combine_body,
        out_shape=jax.ShapeDtypeStruct((N, D_OUT), jnp.float32))(x1o, g)


@jax.jit
def kernel(x1, edge_index1, x2, edge_index2, group_assignment,
           W1, att_src1, att_dst1, b1, W2, att_src2, att_dst2, b2):
    xp1, xp2, as1, ad1, m1, as2, ad2, m2 = _dense_call(
        x1, x2, W1, W2, att_src1, att_dst1, att_src2, att_dst2)

    loop = jnp.arange(N, dtype=jnp.int32)
    pad = jnp.zeros((E_PAD - E_VALID,), dtype=jnp.int32)
    src1 = jnp.concatenate([edge_index1[0], loop, pad])
    dst1 = jnp.concatenate([edge_index1[1], loop, pad])
    src2 = jnp.concatenate([edge_index2[0], loop, pad])
    dst2 = jnp.concatenate([edge_index2[1], loop, pad])

    flat = lambda a: a.reshape(-1)
    ou1, dn1, ou2, dn2 = _edge_call(
        src1, dst1, src2, dst2,
        flat(as1), flat(ad1), flat(m1), flat(as2), flat(ad2), flat(m2),
        xp1, xp2)

    x1_out, x2_out = _norm_call(ou1, dn1, ou2, dn2, b1, b2)

    grp = jnp.concatenate(
        [group_assignment.astype(jnp.int32),
         jnp.zeros((N_PAD - N,), dtype=jnp.int32)])
    g_rows = _gather_call(x2_out, grp)

    x1_combined = _combine_call(x1_out[:N], g_rows[:N])
    return (x1_combined, x2_out[:N])


# speedup vs baseline: 33.4720x; 33.4720x over previous
import jax
import jax.numpy as jnp
from jax import lax
from jax.experimental import pallas as pl
from jax.experimental.pallas import tpu as pltpu
from jax.experimental.pallas import tpu_sc as plsc

N = 10000
E = 320000
D_IN = 128
D_OUT = 16

NC = 2
NS = 16
L = 16

E_VALID = E + N
CHUNK = 128
NW = NC * NS
K_CHUNKS = -(-E_VALID // (CHUNK * NW))
E_TILE = K_CHUNKS * CHUNK
E_PAD = E_TILE * NW
N_PAD = 10240
ROWS_T = N_PAD // NS
G_TILE = N_PAD // NW
G_CHUNK = 64
G_STEPS = G_TILE // G_CHUNK


def _leaky(x):
    return jnp.where(x >= 0, x, 0.2 * x)


def _dense_body(x1, x2, w1, w2, s1, d1, s2, d2,
                xp1_o, xp2_o, as1_o, ad1_o, m1_o, as2_o, ad2_o, m2_o):
    xp1 = jnp.dot(x1[...], w1[...], preferred_element_type=jnp.float32)
    xp2 = jnp.dot(x2[...], w2[...], preferred_element_type=jnp.float32)
    xp1_o[...] = xp1
    xp2_o[...] = xp2
    as1 = jnp.sum(xp1 * s1[...], axis=1, keepdims=True)
    ad1 = jnp.sum(xp1 * d1[...], axis=1, keepdims=True)
    as2 = jnp.sum(xp2 * s2[...], axis=1, keepdims=True)
    ad2 = jnp.sum(xp2 * d2[...], axis=1, keepdims=True)
    as1_o[...] = as1
    ad1_o[...] = ad1
    as2_o[...] = as2
    ad2_o[...] = ad2
    m1_o[...] = _leaky(jnp.max(as1) + ad1)
    m2_o[...] = _leaky(jnp.max(as2) + ad2)


def _dense_call(x1, x2, w1, w2, s1, d1, s2, d2):
    f32 = jnp.float32
    out_shape = [jax.ShapeDtypeStruct((N, D_OUT), f32)] * 2 + \
                [jax.ShapeDtypeStruct((N, 1), f32)] * 6
    return pl.pallas_call(_dense_body, out_shape=out_shape)(
        x1, x2, w1, w2, s1.reshape(1, D_OUT), d1.reshape(1, D_OUT),
        s2.reshape(1, D_OUT), d2.reshape(1, D_OUT))


def _edges_one_gat(c, s, src_h, dst_h, as_h, ad_h, m_h, xp_h, ou_h, dn_h,
                   as_t, ad_t, m_t, den_t, zrow, src_c, dst_c, e_c, rows,
                   acc, sem):
    iota16 = lax.iota(jnp.int32, L)

    pltpu.sync_copy(as_h, as_t)
    pltpu.sync_copy(ad_h, ad_t)
    pltpu.sync_copy(m_h, m_t)

    def _zero_den(j, _):
        den_t[pl.ds(j * L, L)] = jnp.zeros((L,), jnp.float32)
        return 0
    lax.fori_loop(0, N_PAD // L, _zero_den, 0)
    pltpu.sync_copy(zrow, acc.at[pl.ds(s * ROWS_T, ROWS_T)])
    plsc.subcore_barrier()

    base0 = c * (NS * E_TILE) + s * E_TILE

    def _chunk(k, _):
        base = base0 + k * CHUNK
        pltpu.sync_copy(src_h.at[pl.ds(base, CHUNK)], src_c)
        pltpu.sync_copy(dst_h.at[pl.ds(base, CHUNK)], dst_c)
        row_cp = pltpu.async_copy(xp_h.at[src_c], rows, sem)

        def _evec(j, _):
            si = src_c[pl.ds(j * L, L)]
            di = dst_c[pl.ds(j * L, L)]
            a = plsc.load_gather(as_t, [si]) + plsc.load_gather(ad_t, [di])
            a = jnp.where(a >= 0, a, 0.2 * a)
            e = jnp.exp(a - plsc.load_gather(m_t, [di]))
            gid = base + j * L + iota16
            e = jnp.where(gid < E_VALID, e, 0.0)
            e_c[pl.ds(j * L, L)] = e
            plsc.addupdate_scatter(den_t, [di], e)
            return 0
        lax.fori_loop(0, CHUNK // L, _evec, 0, unroll=True)

        row_cp.wait()

        def _scale(j, _):
            w = e_c[pl.ds(j, L)][0]
            rows[j, :] = rows[j, :] * w
            return 0
        lax.fori_loop(0, CHUNK, _scale, 0, unroll=8)

        pltpu.sync_copy(rows, acc.at[dst_c], add=True)
        return 0

    lax.fori_loop(0, K_CHUNKS, _chunk, 0)
    plsc.subcore_barrier()

    pltpu.sync_copy(acc.at[pl.ds(s * ROWS_T, ROWS_T)],
                    ou_h.at[c, pl.ds(s * ROWS_T, ROWS_T)])
    pltpu.sync_copy(den_t, dn_h.at[c, s])
    plsc.subcore_barrier()


def _edge_body(src1, dst1, src2, dst2, as1, ad1, m1, as2, ad2, m2, xp1, xp2,
               ou1, dn1, ou2, dn2,
               as_t, ad_t, m_t, den_t, zrow, src_c, dst_c, e_c, rows, acc,
               sem):
    c = lax.axis_index("c")
    s = lax.axis_index("s")

    def _zfill(j, _):
        zrow[j, :] = jnp.zeros((L,), jnp.float32)
        return 0
    lax.fori_loop(0, ROWS_T, _zfill, 0)

    _edges_one_gat(c, s, src1, dst1, as1, ad1, m1, xp1, ou1, dn1,
                   as_t, ad_t, m_t, den_t, zrow, src_c, dst_c, e_c, rows,
                   acc, sem)
    _edges_one_gat(c, s, src2, dst2, as2, ad2, m2, xp2, ou2, dn2,
                   as_t, ad_t, m_t, den_t, zrow, src_c, dst_c, e_c, rows,
                   acc, sem)


def _edge_call(src1, dst1, src2, dst2, as1, ad1, m1, as2, ad2, m2, xp1, xp2):
    f32 = jnp.float32
    mesh = plsc.VectorSubcoreMesh(core_axis_name="c", subcore_axis_name="s")
    out_type = [
        jax.ShapeDtypeStruct((NC, N_PAD, D_OUT), f32),
        jax.ShapeDtypeStruct((NC, NS, N_PAD), f32),
        jax.ShapeDtypeStruct((NC, N_PAD, D_OUT), f32),
        jax.ShapeDtypeStruct((NC, NS, N_PAD), f32),
    ]
    scratch = [
        pltpu.VMEM((N,), f32),
        pltpu.VMEM((N,), f32),
        pltpu.VMEM((N,), f32),
        pltpu.VMEM((N_PAD,), f32),
        pltpu.VMEM((ROWS_T, D_OUT), f32),
        pltpu.VMEM((CHUNK,), jnp.int32),
        pltpu.VMEM((CHUNK,), jnp.int32),
        pltpu.VMEM((CHUNK + L,), f32),
        pltpu.VMEM((CHUNK, D_OUT), f32),
        pltpu.VMEM_SHARED((N_PAD, D_OUT), f32),
        pltpu.SemaphoreType.DMA,
    ]
    kfn = pl.kernel(_edge_body, out_type=out_type, mesh=mesh,
                    scratch_types=scratch,
                    compiler_params=pltpu.CompilerParams(
                        needs_layout_passes=False,
                        use_tc_tiling_on_sc=False))
    return kfn(src1, dst1, src2, dst2, as1, ad1, m1, as2, ad2, m2, xp1, xp2)


def _norm_body(ou1, dn1, ou2, dn2, b1, b2, x1o, x2o):
    s1 = ou1[0] + ou1[1]
    s2 = ou2[0] + ou2[1]
    d1 = jnp.sum(dn1[...], axis=(0, 1))[:, None]
    d2 = jnp.sum(dn2[...], axis=(0, 1))[:, None]
    x1o[...] = s1 / d1 + b1[...]
    x2o[...] = s2 / d2 + b2[...]


def _norm_call(ou1, dn1, ou2, dn2, b1, b2):
    f32 = jnp.float32
    out_shape = [jax.ShapeDtypeStruct((N_PAD, D_OUT), f32)] * 2
    return pl.pallas_call(_norm_body, out_shape=out_shape)(
        ou1, dn1, ou2, dn2, b1.reshape(1, D_OUT), b2.reshape(1, D_OUT))


def _gather_body(x2o, grp, out, idx_v, rows_v, sem):
    c = lax.axis_index("c")
    s = lax.axis_index("s")
    base = (s * NC + c) * G_TILE

    def _step(k, _):
        off = base + k * G_CHUNK
        pltpu.sync_copy(grp.at[pl.ds(off, G_CHUNK)], idx_v)
        pltpu.async_copy(x2o.at[idx_v], rows_v, sem).wait()
        pltpu.sync_copy(rows_v, out.at[pl.ds(off, G_CHUNK)])
        return 0
    lax.fori_loop(0, G_STEPS, _step, 0)


def _gather_call(x2o, grp):
    mesh = plsc.VectorSubcoreMesh(core_axis_name="c", subcore_axis_name="s")
    out_type = jax.ShapeDtypeStruct((N_PAD, D_OUT), jnp.float32)
    scratch = [
        pltpu.VMEM((G_CHUNK,), jnp.int32),
        pltpu.VMEM((G_CHUNK, D_OUT), jnp.float32),
        pltpu.SemaphoreType.DMA,
    ]
    kfn = pl.kernel(_gather_body, out_type=out_type, mesh=mesh,
                    scratch_types=scratch,
                    compiler_params=pltpu.CompilerParams(
                        needs_layout_passes=False,
                        use_tc_tiling_on_sc=False))
    return kfn(x2o, grp)


def _combine_body(x1o, g, out):
    out[...] = x1o[...] + g[...]


def _combine_call(x1o, g):
    return pl.pallas_call(
        _combine_body,
        out_shape=jax.ShapeDtypeStruct((N, D_OUT), jnp.float32))(x1o, g)


@jax.jit
def kernel(x1, edge_index1, x2, edge_index2, group_assignment,
           W1, att_src1, att_dst1, b1, W2, att_src2, att_dst2, b2):
    xp1, xp2, as1, ad1, m1, as2, ad2, m2 = _dense_call(
        x1, x2, W1, W2, att_src1, att_dst1, att_src2, att_dst2)

    loop = jnp.arange(N, dtype=jnp.int32)
    pad = jnp.zeros((E_PAD - E_VALID,), dtype=jnp.int32)
    src1 = jnp.concatenate([edge_index1[0], loop, pad])
    dst1 = jnp.concatenate([edge_index1[1], loop, pad])
    src2 = jnp.concatenate([edge_index2[0], loop, pad])
    dst2 = jnp.concatenate([edge_index2[1], loop, pad])

    flat = lambda a: a.reshape(-1)
    ou1, dn1, ou2, dn2 = _edge_call(
        src1, dst1, src2, dst2,
        flat(as1), flat(ad1), flat(m1), flat(as2), flat(ad2), flat(m2),
        xp1, xp2)

    x1_out, x2_out = _norm_call(ou1, dn1, ou2, dn2, b1, b2)

    grp = jnp.concatenate(
        [group_assignment.astype(jnp.int32),
         jnp.zeros((N_PAD - N,), dtype=jnp.int32)])
    g_rows = _gather_call(x2_out, grp)

    x1_combined = _combine_call(x1_out[:N], g_rows[:N])
    return (x1_combined, x2_out[:N])


# speedup vs baseline: 54.5085x; 1.6285x over previous
import jax
import jax.numpy as jnp
from jax import lax
from jax.experimental import pallas as pl
from jax.experimental.pallas import tpu as pltpu
from jax.experimental.pallas import tpu_sc as plsc

N = 10000
E = 320000
D_IN = 128
D_OUT = 16

NC = 2
NS = 16
L = 16

E_VALID = E + N
CHUNK = 128
NW = NC * NS
K_CHUNKS = -(-E_VALID // (CHUNK * NW))
K_CHUNKS += K_CHUNKS % 2
E_TILE = K_CHUNKS * CHUNK
E_PAD = E_TILE * NW
N_PAD = 10240
ROWS_T = N_PAD // NS
G_TILE = N_PAD // NW
G_CHUNK = 64
G_STEPS = G_TILE // G_CHUNK


def _leaky(x):
    return jnp.where(x >= 0, x, 0.2 * x)


def _dense_body(x1, x2, w1, w2, s1, d1, s2, d2,
                xp1_o, xp2_o, as1_o, ad1_o, m1_o, as2_o, ad2_o, m2_o):
    xp1 = jnp.dot(x1[...], w1[...], preferred_element_type=jnp.float32)
    xp2 = jnp.dot(x2[...], w2[...], preferred_element_type=jnp.float32)
    xp1_o[...] = xp1
    xp2_o[...] = xp2
    as1 = jnp.sum(xp1 * s1[...], axis=1, keepdims=True)
    ad1 = jnp.sum(xp1 * d1[...], axis=1, keepdims=True)
    as2 = jnp.sum(xp2 * s2[...], axis=1, keepdims=True)
    ad2 = jnp.sum(xp2 * d2[...], axis=1, keepdims=True)
    as1_o[...] = as1
    ad1_o[...] = ad1
    as2_o[...] = as2
    ad2_o[...] = ad2
    m1_o[...] = _leaky(jnp.max(as1) + ad1)
    m2_o[...] = _leaky(jnp.max(as2) + ad2)


def _dense_call(x1, x2, w1, w2, s1, d1, s2, d2):
    f32 = jnp.float32
    out_shape = [jax.ShapeDtypeStruct((N, D_OUT), f32)] * 2 + \
                [jax.ShapeDtypeStruct((N, 1), f32)] * 6
    return pl.pallas_call(_dense_body, out_shape=out_shape)(
        x1, x2, w1, w2, s1.reshape(1, D_OUT), d1.reshape(1, D_OUT),
        s2.reshape(1, D_OUT), d2.reshape(1, D_OUT))


def _edges_one_gat(c, s, src_h, dst_h, as_h, ad_h, m_h, xp_h, ou_h, dn_h,
                   as_t, ad_t, m_t, den_t, zrow,
                   src_b, dst_b, e_c, rows_b, acc, sem_i, sem_g):
    iota16 = lax.iota(jnp.int32, L)

    pltpu.sync_copy(as_h, as_t)
    pltpu.sync_copy(ad_h, ad_t)
    pltpu.sync_copy(m_h, m_t)

    def _zero_den(j, _):
        den_t[pl.ds(j * L, L)] = jnp.zeros((L,), jnp.float32)
        return 0
    lax.fori_loop(0, N_PAD // L, _zero_den, 0)
    pltpu.sync_copy(zrow, acc.at[pl.ds(s * ROWS_T, ROWS_T)])
    plsc.subcore_barrier()

    base0 = c * (NS * E_TILE) + s * E_TILE

    def drain_idx(p):
        pltpu.make_async_copy(src_h.at[pl.ds(0, CHUNK)], src_b[p],
                              sem_i).wait()
        pltpu.make_async_copy(src_h.at[pl.ds(0, CHUNK)], dst_b[p],
                              sem_i).wait()

    def phase(k, p, issue_next, prefetch_idx):
        if issue_next:
            drain_idx(1 - p)
            pltpu.async_copy(xp_h.at[src_b[1 - p]], rows_b[1 - p],
                             sem_g[1 - p])

        base = base0 + k * CHUNK

        def _evec(j, _):
            si = src_b[p][pl.ds(j * L, L)]
            di = dst_b[p][pl.ds(j * L, L)]
            a = plsc.load_gather(as_t, [si]) + plsc.load_gather(ad_t, [di])
            a = jnp.where(a >= 0, a, 0.2 * a)
            e = jnp.exp(a - plsc.load_gather(m_t, [di]))
            gid = base + j * L + iota16
            e = jnp.where(gid < E_VALID, e, 0.0)
            e_c[pl.ds(j * L, L)] = e
            plsc.addupdate_scatter(den_t, [di], e)
            return 0
        lax.fori_loop(0, CHUNK // L, _evec, 0, unroll=True)

        pltpu.make_async_copy(xp_h.at[pl.ds(0, CHUNK)], rows_b[p],
                              sem_g[p]).wait()

        def _scale(jj, _):
            w16 = e_c[pl.ds(jj * L, L)]
            for i in range(L):
                r = jj * L + i
                rows_b[p][r, :] = rows_b[p][r, :] * w16[i]
            return 0
        lax.fori_loop(0, CHUNK // L, _scale, 0)

        pltpu.sync_copy(rows_b[p], acc.at[dst_b[p]], add=True)

        if prefetch_idx:
            b2 = base + 2 * CHUNK
            pltpu.async_copy(src_h.at[pl.ds(b2, CHUNK)], src_b[p], sem_i)
            pltpu.async_copy(dst_h.at[pl.ds(b2, CHUNK)], dst_b[p], sem_i)

    pltpu.sync_copy(src_h.at[pl.ds(base0, CHUNK)], src_b[0])
    pltpu.sync_copy(dst_h.at[pl.ds(base0, CHUNK)], dst_b[0])
    pltpu.async_copy(xp_h.at[src_b[0]], rows_b[0], sem_g[0])
    pltpu.async_copy(src_h.at[pl.ds(base0 + CHUNK, CHUNK)], src_b[1], sem_i)
    pltpu.async_copy(dst_h.at[pl.ds(base0 + CHUNK, CHUNK)], dst_b[1], sem_i)

    def _pair(k2, _):
        k = k2 * 2
        phase(k, 0, True, True)
        phase(k + 1, 1, True, True)
        return 0
    lax.fori_loop(0, K_CHUNKS // 2 - 1, _pair, 0)
    phase(K_CHUNKS - 2, 0, True, False)
    phase(K_CHUNKS - 1, 1, False, False)

    plsc.subcore_barrier()

    pltpu.sync_copy(acc.at[pl.ds(s * ROWS_T, ROWS_T)],
                    ou_h.at[c, pl.ds(s * ROWS_T, ROWS_T)])
    pltpu.sync_copy(den_t, dn_h.at[c, s])
    plsc.subcore_barrier()


def _edge_body(src1, dst1, src2, dst2, as1, ad1, m1, as2, ad2, m2, xp1, xp2,
               ou1, dn1, ou2, dn2,
               as_t, ad_t, m_t, den_t, zrow,
               src_c0, src_c1, dst_c0, dst_c1, e_c, rows0, rows1, acc,
               sem_i, sem_g0, sem_g1):
    c = lax.axis_index("c")
    s = lax.axis_index("s")

    def _zfill(j, _):
        zrow[j, :] = jnp.zeros((L,), jnp.float32)
        return 0
    lax.fori_loop(0, ROWS_T, _zfill, 0)

    src_b = (src_c0, src_c1)
    dst_b = (dst_c0, dst_c1)
    rows_b = (rows0, rows1)
    sem_g = (sem_g0, sem_g1)
    _edges_one_gat(c, s, src1, dst1, as1, ad1, m1, xp1, ou1, dn1,
                   as_t, ad_t, m_t, den_t, zrow,
                   src_b, dst_b, e_c, rows_b, acc, sem_i, sem_g)
    _edges_one_gat(c, s, src2, dst2, as2, ad2, m2, xp2, ou2, dn2,
                   as_t, ad_t, m_t, den_t, zrow,
                   src_b, dst_b, e_c, rows_b, acc, sem_i, sem_g)


def _edge_call(src1, dst1, src2, dst2, as1, ad1, m1, as2, ad2, m2, xp1, xp2):
    f32 = jnp.float32
    mesh = plsc.VectorSubcoreMesh(core_axis_name="c", subcore_axis_name="s")
    out_type = [
        jax.ShapeDtypeStruct((NC, N_PAD, D_OUT), f32),
        jax.ShapeDtypeStruct((NC, NS, N_PAD), f32),
        jax.ShapeDtypeStruct((NC, N_PAD, D_OUT), f32),
        jax.ShapeDtypeStruct((NC, NS, N_PAD), f32),
    ]
    scratch = [
        pltpu.VMEM((N,), f32),
        pltpu.VMEM((N,), f32),
        pltpu.VMEM((N,), f32),
        pltpu.VMEM((N_PAD,), f32),
        pltpu.VMEM((ROWS_T, D_OUT), f32),
        pltpu.VMEM((CHUNK,), jnp.int32),
        pltpu.VMEM((CHUNK,), jnp.int32),
        pltpu.VMEM((CHUNK,), jnp.int32),
        pltpu.VMEM((CHUNK,), jnp.int32),
        pltpu.VMEM((CHUNK,), f32),
        pltpu.VMEM((CHUNK, D_OUT), f32),
        pltpu.VMEM((CHUNK, D_OUT), f32),
        pltpu.VMEM_SHARED((N_PAD, D_OUT), f32),
        pltpu.SemaphoreType.DMA,
        pltpu.SemaphoreType.DMA,
        pltpu.SemaphoreType.DMA,
    ]
    kfn = pl.kernel(_edge_body, out_type=out_type, mesh=mesh,
                    scratch_types=scratch,
                    compiler_params=pltpu.CompilerParams(
                        needs_layout_passes=False,
                        use_tc_tiling_on_sc=False))
    return kfn(src1, dst1, src2, dst2, as1, ad1, m1, as2, ad2, m2, xp1, xp2)


def _norm_body(ou1, dn1, ou2, dn2, b1, b2, x1o, x2o):
    s1 = ou1[0] + ou1[1]
    s2 = ou2[0] + ou2[1]
    d1 = jnp.sum(dn1[...], axis=(0, 1))[:, None]
    d2 = jnp.sum(dn2[...], axis=(0, 1))[:, None]
    x1o[...] = s1 / d1 + b1[...]
    x2o[...] = s2 / d2 + b2[...]


def _norm_call(ou1, dn1, ou2, dn2, b1, b2):
    f32 = jnp.float32
    out_shape = [jax.ShapeDtypeStruct((N_PAD, D_OUT), f32)] * 2
    return pl.pallas_call(_norm_body, out_shape=out_shape)(
        ou1, dn1, ou2, dn2, b1.reshape(1, D_OUT), b2.reshape(1, D_OUT))


def _gather_body(x2o, grp, out, idx_v, rows_v, sem):
    c = lax.axis_index("c")
    s = lax.axis_index("s")
    base = (s * NC + c) * G_TILE

    def _step(k, _):
        off = base + k * G_CHUNK
        pltpu.sync_copy(grp.at[pl.ds(off, G_CHUNK)], idx_v)
        pltpu.async_copy(x2o.at[idx_v], rows_v, sem).wait()
        pltpu.sync_copy(rows_v, out.at[pl.ds(off, G_CHUNK)])
        return 0
    lax.fori_loop(0, G_STEPS, _step, 0)


def _gather_call(x2o, grp):
    mesh = plsc.VectorSubcoreMesh(core_axis_name="c", subcore_axis_name="s")
    out_type = jax.ShapeDtypeStruct((N_PAD, D_OUT), jnp.float32)
    scratch = [
        pltpu.VMEM((G_CHUNK,), jnp.int32),
        pltpu.VMEM((G_CHUNK, D_OUT), jnp.float32),
        pltpu.SemaphoreType.DMA,
    ]
    kfn = pl.kernel(_gather_body, out_type=out_type, mesh=mesh,
                    scratch_types=scratch,
                    compiler_params=pltpu.CompilerParams(
                        needs_layout_passes=False,
                        use_tc_tiling_on_sc=False))
    return kfn(x2o, grp)


def _combine_body(x1o, g, out):
    out[...] = x1o[...] + g[...]


def _combine_call(x1o, g):
    return pl.pallas_call(
        _combine_body,
        out_shape=jax.ShapeDtypeStruct((N, D_OUT), jnp.float32))(x1o, g)


@jax.jit
def kernel(x1, edge_index1, x2, edge_index2, group_assignment,
           W1, att_src1, att_dst1, b1, W2, att_src2, att_dst2, b2):
    xp1, xp2, as1, ad1, m1, as2, ad2, m2 = _dense_call(
        x1, x2, W1, W2, att_src1, att_dst1, att_src2, att_dst2)

    loop = jnp.arange(N, dtype=jnp.int32)
    pad = jnp.zeros((E_PAD - E_VALID,), dtype=jnp.int32)
    src1 = jnp.concatenate([edge_index1[0], loop, pad])
    dst1 = jnp.concatenate([edge_index1[1], loop, pad])
    src2 = jnp.concatenate([edge_index2[0], loop, pad])
    dst2 = jnp.concatenate([edge_index2[1], loop, pad])

    flat = lambda a: a.reshape(-1)
    ou1, dn1, ou2, dn2 = _edge_call(
        src1, dst1, src2, dst2,
        flat(as1), flat(ad1), flat(m1), flat(as2), flat(ad2), flat(m2),
        xp1, xp2)

    x1_out, x2_out = _norm_call(ou1, dn1, ou2, dn2, b1, b2)

    grp = jnp.concatenate(
        [group_assignment.astype(jnp.int32),
         jnp.zeros((N_PAD - N,), dtype=jnp.int32)])
    g_rows = _gather_call(x2_out, grp)

    x1_combined = _combine_call(x1_out[:N], g_rows[:N])
    return (x1_combined, x2_out[:N])


# speedup vs baseline: 75.2982x; 1.3814x over previous
import jax
import jax.numpy as jnp
from jax import lax
from jax.experimental import pallas as pl
from jax.experimental.pallas import tpu as pltpu
from jax.experimental.pallas import tpu_sc as plsc

N = 10000
E = 320000
D_IN = 128
D_OUT = 16

NC = 2
NS = 16
L = 16

CHUNK = 128
K_CHUNKS = -(-E // (CHUNK * NS))
K_CHUNKS += K_CHUNKS % 2
E_TILE = K_CHUNKS * CHUNK
E_PAD = E_TILE * NS
N_PAD = 10240
ROWS_T = N_PAD // NS
G_TILE = N_PAD // (NC * NS)
G_CHUNK = 64
G_STEPS = G_TILE // G_CHUNK


def _leaky(x):
    return jnp.where(x >= 0, x, 0.2 * x)


def _dense_body(x1, x2, w1, w2, s1, d1, s2, d2,
                xp1_o, xp2_o, as1_o, ad1_o, m1_o, as2_o, ad2_o, m2_o):
    xp1 = jnp.dot(x1[...], w1[...], preferred_element_type=jnp.float32)
    xp2 = jnp.dot(x2[...], w2[...], preferred_element_type=jnp.float32)
    xp1_o[...] = xp1
    xp2_o[...] = xp2
    as1 = jnp.sum(xp1 * s1[...], axis=1, keepdims=True)
    ad1 = jnp.sum(xp1 * d1[...], axis=1, keepdims=True)
    as2 = jnp.sum(xp2 * s2[...], axis=1, keepdims=True)
    ad2 = jnp.sum(xp2 * d2[...], axis=1, keepdims=True)
    as1_o[...] = as1
    ad1_o[...] = ad1
    as2_o[...] = as2
    ad2_o[...] = ad2
    m1_o[...] = _leaky(jnp.max(as1) + ad1)
    m2_o[...] = _leaky(jnp.max(as2) + ad2)


def _dense_call(x1, x2, w1, w2, s1, d1, s2, d2):
    f32 = jnp.float32
    out_shape = [jax.ShapeDtypeStruct((N, D_OUT), f32)] * 2 + \
                [jax.ShapeDtypeStruct((N, 1), f32)] * 6
    return pl.pallas_call(_dense_body, out_shape=out_shape)(
        x1, x2, w1, w2, s1.reshape(1, D_OUT), d1.reshape(1, D_OUT),
        s2.reshape(1, D_OUT), d2.reshape(1, D_OUT))


def _gat_on_core(s, ei_h, as_h, ad_h, m_h, xp_h, b_h, out_h,
                 as_t, ad_t, m_t, idx2_b, e_b, rows_b, rbuf, den_v, bias_v,
                 acc, den_acc, sem_i, sem_g):
    iota16 = lax.iota(jnp.int32, L)

    pltpu.sync_copy(as_h, as_t)
    pltpu.sync_copy(ad_h, ad_t)
    pltpu.sync_copy(m_h, m_t)
    pltpu.sync_copy(b_h, bias_v)

    vbase = s * ROWS_T
    pltpu.sync_copy(xp_h.at[pl.ds(vbase, ROWS_T)], rbuf)

    def _self(jj, _):
        v0 = vbase + jj * L
        a = as_t[pl.ds(v0, L)] + ad_t[pl.ds(v0, L)]
        a = jnp.where(a >= 0, a, 0.2 * a)
        ev = jnp.exp(a - m_t[pl.ds(v0, L)])
        den_v[pl.ds(jj * L, L)] = ev
        for i in range(L):
            r = jj * L + i
            rbuf[r, :] = rbuf[r, :] * ev[i]
        return 0
    lax.fori_loop(0, ROWS_T // L, _self, 0)
    pltpu.sync_copy(rbuf, acc.at[pl.ds(vbase, ROWS_T)])
    pltpu.sync_copy(den_v, den_acc.at[pl.ds(vbase, ROWS_T)])
    plsc.subcore_barrier()

    base0 = s * E_TILE

    def phase(k, p, issue_next, prefetch_idx):
        if issue_next:
            pltpu.make_async_copy(ei_h.at[:, pl.ds(0, CHUNK)], idx2_b[1 - p],
                                  sem_i).wait()
            pltpu.async_copy(xp_h.at[idx2_b[1 - p].at[0]], rows_b[1 - p],
                             sem_g[1 - p])

        base = base0 + k * CHUNK

        def _evec(j, _):
            si = idx2_b[p][0, pl.ds(j * L, L)]
            di = idx2_b[p][1, pl.ds(j * L, L)]
            a = plsc.load_gather(as_t, [si]) + plsc.load_gather(ad_t, [di])
            a = jnp.where(a >= 0, a, 0.2 * a)
            e = jnp.exp(a - plsc.load_gather(m_t, [di]))
            gid = base + j * L + iota16
            e = jnp.where(gid < E, e, 0.0)
            e_b[p][pl.ds(j * L, L)] = e
            return 0
        lax.fori_loop(0, CHUNK // L, _evec, 0, unroll=True)
        pltpu.sync_copy(e_b[p], den_acc.at[idx2_b[p].at[1]], add=True)

        pltpu.make_async_copy(xp_h.at[pl.ds(0, CHUNK)], rows_b[p],
                              sem_g[p]).wait()

        def _scale(jj, _):
            w16 = e_b[p][pl.ds(jj * L, L)]
            for i in range(L):
                r = jj * L + i
                rows_b[p][r, :] = rows_b[p][r, :] * w16[i]
            return 0
        lax.fori_loop(0, CHUNK // L, _scale, 0)
        pltpu.sync_copy(rows_b[p], acc.at[idx2_b[p].at[1]], add=True)

        if prefetch_idx:
            pltpu.async_copy(ei_h.at[:, pl.ds(base + 2 * CHUNK, CHUNK)],
                             idx2_b[p], sem_i)

    pltpu.sync_copy(ei_h.at[:, pl.ds(base0, CHUNK)], idx2_b[0])
    pltpu.async_copy(xp_h.at[idx2_b[0].at[0]], rows_b[0], sem_g[0])
    pltpu.async_copy(ei_h.at[:, pl.ds(base0 + CHUNK, CHUNK)], idx2_b[1],
                     sem_i)

    def _pair(k2, _):
        k = k2 * 2
        phase(k, 0, True, True)
        phase(k + 1, 1, True, True)
        return 0
    lax.fori_loop(0, K_CHUNKS // 2 - 1, _pair, 0)
    phase(K_CHUNKS - 2, 0, True, False)
    phase(K_CHUNKS - 1, 1, False, False)

    plsc.subcore_barrier()

    pltpu.sync_copy(acc.at[pl.ds(vbase, ROWS_T)], rbuf)
    pltpu.sync_copy(den_acc.at[pl.ds(vbase, ROWS_T)], den_v)
    bias = bias_v[...]

    def _norm(jj, _):
        inv = 1.0 / den_v[pl.ds(jj * L, L)]
        for i in range(L):
            r = jj * L + i
            rbuf[r, :] = rbuf[r, :] * inv[i] + bias
        return 0
    lax.fori_loop(0, ROWS_T // L, _norm, 0)
    pltpu.sync_copy(rbuf, out_h.at[pl.ds(vbase, ROWS_T)])


def _edge_body(ei1, ei2, as1, ad1, m1, as2, ad2, m2, xp1, xp2, b1, b2,
               x1o, x2o,
               as_t, ad_t, m_t, idx2_0, idx2_1, e_c0, e_c1, rows0, rows1,
               rbuf, den_v, bias_v, acc, den_acc, sem_i, sem_g0, sem_g1):
    c = lax.axis_index("c")
    s = lax.axis_index("s")
    idx2_b = (idx2_0, idx2_1)
    e_b = (e_c0, e_c1)
    rows_b = (rows0, rows1)
    sem_g = (sem_g0, sem_g1)

    @pl.when(c == 0)
    def _():
        _gat_on_core(s, ei1, as1, ad1, m1, xp1, b1, x1o,
                     as_t, ad_t, m_t, idx2_b, e_b, rows_b, rbuf, den_v,
                     bias_v, acc, den_acc, sem_i, sem_g)

    @pl.when(c == 1)
    def _():
        _gat_on_core(s, ei2, as2, ad2, m2, xp2, b2, x2o,
                     as_t, ad_t, m_t, idx2_b, e_b, rows_b, rbuf, den_v,
                     bias_v, acc, den_acc, sem_i, sem_g)


def _edge_call(ei1, ei2, as1, ad1, m1, as2, ad2, m2, xp1, xp2, b1, b2):
    f32 = jnp.float32
    mesh = plsc.VectorSubcoreMesh(core_axis_name="c", subcore_axis_name="s")
    out_type = [
        jax.ShapeDtypeStruct((N_PAD, D_OUT), f32),
        jax.ShapeDtypeStruct((N_PAD, D_OUT), f32),
    ]
    scratch = [
        pltpu.VMEM((N_PAD,), f32),
        pltpu.VMEM((N_PAD,), f32),
        pltpu.VMEM((N_PAD,), f32),
        pltpu.VMEM((2, CHUNK), jnp.int32),
        pltpu.VMEM((2, CHUNK), jnp.int32),
        pltpu.VMEM((CHUNK,), f32),
        pltpu.VMEM((CHUNK,), f32),
        pltpu.VMEM((CHUNK, D_OUT), f32),
        pltpu.VMEM((CHUNK, D_OUT), f32),
        pltpu.VMEM((ROWS_T, D_OUT), f32),
        pltpu.VMEM((ROWS_T,), f32),
        pltpu.VMEM((L,), f32),
        pltpu.VMEM_SHARED((N_PAD, D_OUT), f32),
        pltpu.VMEM_SHARED((N_PAD,), f32),
        pltpu.SemaphoreType.DMA,
        pltpu.SemaphoreType.DMA,
        pltpu.SemaphoreType.DMA,
    ]
    kfn = pl.kernel(_edge_body, out_type=out_type, mesh=mesh,
                    scratch_types=scratch,
                    compiler_params=pltpu.CompilerParams(
                        needs_layout_passes=False,
                        use_tc_tiling_on_sc=False))
    return kfn(ei1, ei2, as1, ad1, m1, as2, ad2, m2, xp1, xp2, b1, b2)


def _gather_body(x1o, x2o, grp, out, idx_v, gbuf, xbuf, sem):
    c = lax.axis_index("c")
    s = lax.axis_index("s")
    base = (s * NC + c) * G_TILE

    def _step(k, _):
        off = base + k * G_CHUNK
        pltpu.sync_copy(grp.at[pl.ds(off, G_CHUNK)], idx_v)
        cp = pltpu.async_copy(x2o.at[idx_v], gbuf, sem)
        pltpu.sync_copy(x1o.at[pl.ds(off, G_CHUNK)], xbuf)
        cp.wait()

        def _add(j, _):
            xbuf[j, :] = xbuf[j, :] + gbuf[j, :]
            return 0
        lax.fori_loop(0, G_CHUNK, _add, 0, unroll=8)
        pltpu.sync_copy(xbuf, out.at[pl.ds(off, G_CHUNK)])
        return 0
    lax.fori_loop(0, G_STEPS, _step, 0)


def _gather_call(x1o, x2o, grp):
    mesh = plsc.VectorSubcoreMesh(core_axis_name="c", subcore_axis_name="s")
    out_type = jax.ShapeDtypeStruct((N_PAD, D_OUT), jnp.float32)
    scratch = [
        pltpu.VMEM((G_CHUNK,), jnp.int32),
        pltpu.VMEM((G_CHUNK, D_OUT), jnp.float32),
        pltpu.VMEM((G_CHUNK, D_OUT), jnp.float32),
        pltpu.SemaphoreType.DMA,
    ]
    kfn = pl.kernel(_gather_body, out_type=out_type, mesh=mesh,
                    scratch_types=scratch,
                    compiler_params=pltpu.CompilerParams(
                        needs_layout_passes=False,
                        use_tc_tiling_on_sc=False))
    return kfn(x1o, x2o, grp)


@jax.jit
def kernel(x1, edge_index1, x2, edge_index2, group_assignment,
           W1, att_src1, att_dst1, b1, W2, att_src2, att_dst2, b2):
    xp1, xp2, as1, ad1, m1, as2, ad2, m2 = _dense_call(
        x1, x2, W1, W2, att_src1, att_dst1, att_src2, att_dst2)

    padt = lambda a: jnp.pad(a.reshape(-1), (0, N_PAD - N))
    padr = lambda a: jnp.pad(a, ((0, N_PAD - N), (0, 0)))
    ei1p = jnp.pad(edge_index1, ((0, 0), (0, E_PAD - E)))
    ei2p = jnp.pad(edge_index2, ((0, 0), (0, E_PAD - E)))

    x1_out, x2_out = _edge_call(
        ei1p, ei2p, padt(as1), padt(ad1), padt(m1),
        padt(as2), padt(ad2), padt(m2), padr(xp1), padr(xp2), b1, b2)

    grp = jnp.pad(group_assignment.astype(jnp.int32), (0, N_PAD - N))
    x1_combined = _gather_call(x1_out, x2_out, grp)
    return (x1_combined[:N], x2_out[:N])


# speedup vs baseline: 76.2219x; 1.0123x over previous
import jax
import jax.numpy as jnp
from jax import lax
from jax.experimental import pallas as pl
from jax.experimental.pallas import tpu as pltpu
from jax.experimental.pallas import tpu_sc as plsc

N = 10000
E = 320000
D_IN = 128
D_OUT = 16

NC = 2
NS = 16
L = 16

CHUNK = 128
K_CHUNKS = -(-E // (CHUNK * NS))
K_CHUNKS += K_CHUNKS % 2
E_TILE = K_CHUNKS * CHUNK
E_PAD = E_TILE * NS
N_PAD = 10240
ROWS_T = N_PAD // NS
G_TILE = N_PAD // (NC * NS)
G_CHUNK = 64
G_STEPS = G_TILE // G_CHUNK


def _leaky(x):
    return jnp.where(x >= 0, x, 0.2 * x)


def _dense_body(x1, x2, w1, w2, s1, d1, s2, d2,
                xp1_o, xp2_o, as1_o, ad1_o, m1_o, as2_o, ad2_o, m2_o):
    xp1 = jnp.dot(x1[...], w1[...], preferred_element_type=jnp.float32)
    xp2 = jnp.dot(x2[...], w2[...], preferred_element_type=jnp.float32)
    xp1_o[...] = xp1
    xp2_o[...] = xp2
    as1 = jnp.sum(xp1 * s1[...], axis=1, keepdims=True)
    ad1 = jnp.sum(xp1 * d1[...], axis=1, keepdims=True)
    as2 = jnp.sum(xp2 * s2[...], axis=1, keepdims=True)
    ad2 = jnp.sum(xp2 * d2[...], axis=1, keepdims=True)
    as1_o[...] = as1
    ad1_o[...] = ad1
    as2_o[...] = as2
    ad2_o[...] = ad2
    m1_o[...] = _leaky(jnp.max(as1) + ad1)
    m2_o[...] = _leaky(jnp.max(as2) + ad2)


def _dense_call(x1, x2, w1, w2, s1, d1, s2, d2):
    f32 = jnp.float32
    out_shape = [jax.ShapeDtypeStruct((N, D_OUT), f32)] * 2 + \
                [jax.ShapeDtypeStruct((N, 1), f32)] * 6
    return pl.pallas_call(_dense_body, out_shape=out_shape)(
        x1, x2, w1, w2, s1.reshape(1, D_OUT), d1.reshape(1, D_OUT),
        s2.reshape(1, D_OUT), d2.reshape(1, D_OUT))


def _gat_on_core(s, ei_h, as_h, ad_h, m_h, xp_h, b_h, out_h,
                 as_t, ad_t, m_t, den_t, den16, idx2_b, e_b, rows_b, rbuf,
                 bias_v, acc, den_stage, sem_i, sem_g):
    iota16 = lax.iota(jnp.int32, L)

    pltpu.sync_copy(as_h, as_t)
    pltpu.sync_copy(ad_h, ad_t)
    pltpu.sync_copy(m_h, m_t)
    pltpu.sync_copy(b_h, bias_v)

    vbase = s * ROWS_T
    pltpu.sync_copy(xp_h.at[pl.ds(vbase, ROWS_T)], rbuf)

    def _zero_den(j, _):
        den_t[pl.ds(j * L, L)] = jnp.zeros((L,), jnp.float32)
        return 0
    lax.fori_loop(0, N_PAD // L, _zero_den, 0)

    def _self(jj, _):
        v0 = vbase + jj * L
        a = as_t[pl.ds(v0, L)] + ad_t[pl.ds(v0, L)]
        a = jnp.where(a >= 0, a, 0.2 * a)
        ev = jnp.exp(a - m_t[pl.ds(v0, L)])
        den_t[pl.ds(v0, L)] = ev
        for i in range(L):
            r = jj * L + i
            rbuf[r, :] = rbuf[r, :] * ev[i]
        return 0
    lax.fori_loop(0, ROWS_T // L, _self, 0)
    pltpu.sync_copy(rbuf, acc.at[pl.ds(vbase, ROWS_T)])
    plsc.subcore_barrier()

    base0 = s * E_TILE

    def phase(k, p, issue_next, prefetch_idx):
        if issue_next:
            pltpu.make_async_copy(ei_h.at[:, pl.ds(0, CHUNK)], idx2_b[1 - p],
                                  sem_i).wait()
            pltpu.async_copy(xp_h.at[idx2_b[1 - p].at[0]], rows_b[1 - p],
                             sem_g[1 - p])

        raw = base0 + k * CHUNK

        def _evec(j, _):
            si = idx2_b[p][0, pl.ds(j * L, L)]
            di = idx2_b[p][1, pl.ds(j * L, L)]
            a = plsc.load_gather(as_t, [si]) + plsc.load_gather(ad_t, [di])
            a = jnp.where(a >= 0, a, 0.2 * a)
            e = jnp.exp(a - plsc.load_gather(m_t, [di]))
            gid = jnp.minimum(raw, E - CHUNK) + j * L + iota16
            e = jnp.where(gid >= raw, e, 0.0)
            e_b[p][pl.ds(j * L, L)] = e
            plsc.addupdate_scatter(den_t, [di], e)
            return 0
        lax.fori_loop(0, CHUNK // L, _evec, 0, unroll=True)

        pltpu.make_async_copy(xp_h.at[pl.ds(0, CHUNK)], rows_b[p],
                              sem_g[p]).wait()

        def _scale(jj, _):
            w16 = e_b[p][pl.ds(jj * L, L)]
            for i in range(L):
                r = jj * L + i
                rows_b[p][r, :] = rows_b[p][r, :] * w16[i]
            return 0
        lax.fori_loop(0, CHUNK // L, _scale, 0)
        pltpu.sync_copy(rows_b[p], acc.at[idx2_b[p].at[1]], add=True)

        if prefetch_idx:
            b2 = jnp.minimum(raw + 2 * CHUNK, E - CHUNK)
            pltpu.async_copy(ei_h.at[:, pl.ds(b2, CHUNK)], idx2_b[p], sem_i)

    pltpu.sync_copy(ei_h.at[:, pl.ds(jnp.minimum(base0, E - CHUNK), CHUNK)],
                    idx2_b[0])
    pltpu.async_copy(xp_h.at[idx2_b[0].at[0]], rows_b[0], sem_g[0])
    pltpu.async_copy(
        ei_h.at[:, pl.ds(jnp.minimum(base0 + CHUNK, E - CHUNK), CHUNK)],
        idx2_b[1], sem_i)

    def _pair(k2, _):
        k = k2 * 2
        phase(k, 0, True, True)
        phase(k + 1, 1, True, True)
        return 0
    lax.fori_loop(0, K_CHUNKS // 2 - 1, _pair, 0)
    phase(K_CHUNKS - 2, 0, True, False)
    phase(K_CHUNKS - 1, 1, False, False)

    pltpu.sync_copy(den_t, den_stage.at[s])
    plsc.subcore_barrier()
    for t in range(NS):
        pltpu.sync_copy(den_stage.at[t, pl.ds(vbase, ROWS_T)], den16.at[t])

    pltpu.sync_copy(acc.at[pl.ds(vbase, ROWS_T)], rbuf)
    bias = bias_v[...]

    def _norm(jj, _):
        d = den16[0, pl.ds(jj * L, L)]
        for t in range(1, NS):
            d = d + den16[t, pl.ds(jj * L, L)]
        inv = 1.0 / d
        for i in range(L):
            r = jj * L + i
            rbuf[r, :] = rbuf[r, :] * inv[i] + bias
        return 0
    lax.fori_loop(0, ROWS_T // L, _norm, 0)
    pltpu.sync_copy(rbuf, out_h.at[pl.ds(vbase, ROWS_T)])


def _edge_body(ei1, ei2, as1, ad1, m1, as2, ad2, m2, xp1, xp2, b1, b2,
               x1o, x2o,
               as_t, ad_t, m_t, den_t, den16, idx2_0, idx2_1, e_c0, e_c1,
               rows0, rows1, rbuf, bias_v, acc, den_stage,
               sem_i, sem_g0, sem_g1):
    c = lax.axis_index("c")
    s = lax.axis_index("s")
    idx2_b = (idx2_0, idx2_1)
    e_b = (e_c0, e_c1)
    rows_b = (rows0, rows1)
    sem_g = (sem_g0, sem_g1)

    @pl.when(c == 0)
    def _():
        _gat_on_core(s, ei1, as1, ad1, m1, xp1, b1, x1o,
                     as_t, ad_t, m_t, den_t, den16, idx2_b, e_b, rows_b,
                     rbuf, bias_v, acc, den_stage, sem_i, sem_g)

    @pl.when(c == 1)
    def _():
        _gat_on_core(s, ei2, as2, ad2, m2, xp2, b2, x2o,
                     as_t, ad_t, m_t, den_t, den16, idx2_b, e_b, rows_b,
                     rbuf, bias_v, acc, den_stage, sem_i, sem_g)


def _edge_call(ei1, ei2, as1, ad1, m1, as2, ad2, m2, xp1, xp2, b1, b2):
    f32 = jnp.float32
    mesh = plsc.VectorSubcoreMesh(core_axis_name="c", subcore_axis_name="s")
    out_type = [
        jax.ShapeDtypeStruct((N_PAD, D_OUT), f32),
        jax.ShapeDtypeStruct((N_PAD, D_OUT), f32),
    ]
    scratch = [
        pltpu.VMEM((N_PAD,), f32),
        pltpu.VMEM((N_PAD,), f32),
        pltpu.VMEM((N_PAD,), f32),
        pltpu.VMEM((N_PAD,), f32),
        pltpu.VMEM((NS, ROWS_T), f32),
        pltpu.VMEM((2, CHUNK), jnp.int32),
        pltpu.VMEM((2, CHUNK), jnp.int32),
        pltpu.VMEM((CHUNK,), f32),
        pltpu.VMEM((CHUNK,), f32),
        pltpu.VMEM((CHUNK, D_OUT), f32),
        pltpu.VMEM((CHUNK, D_OUT), f32),
        pltpu.VMEM((ROWS_T, D_OUT), f32),
        pltpu.VMEM((L,), f32),
        pltpu.VMEM_SHARED((N_PAD, D_OUT), f32),
        pltpu.VMEM_SHARED((NS, N_PAD), f32),
        pltpu.SemaphoreType.DMA,
        pltpu.SemaphoreType.DMA,
        pltpu.SemaphoreType.DMA,
    ]
    kfn = pl.kernel(_edge_body, out_type=out_type, mesh=mesh,
                    scratch_types=scratch,
                    compiler_params=pltpu.CompilerParams(
                        needs_layout_passes=False,
                        use_tc_tiling_on_sc=False))
    return kfn(ei1, ei2, as1, ad1, m1, as2, ad2, m2, xp1, xp2, b1, b2)


def _gather_body(x1o, x2o, grp, out, idx_v, gbuf, xbuf, sem):
    c = lax.axis_index("c")
    s = lax.axis_index("s")
    base = (s * NC + c) * G_TILE

    def _step(k, _):
        off = jnp.minimum(base + k * G_CHUNK, N - G_CHUNK)
        pltpu.sync_copy(grp.at[pl.ds(off, G_CHUNK)], idx_v)
        cp = pltpu.async_copy(x2o.at[idx_v], gbuf, sem)
        pltpu.sync_copy(x1o.at[pl.ds(off, G_CHUNK)], xbuf)
        cp.wait()

        def _add(j, _):
            xbuf[j, :] = xbuf[j, :] + gbuf[j, :]
            return 0
        lax.fori_loop(0, G_CHUNK, _add, 0, unroll=8)
        pltpu.sync_copy(xbuf, out.at[pl.ds(off, G_CHUNK)])
        return 0
    lax.fori_loop(0, G_STEPS, _step, 0)


def _gather_call(x1o, x2o, grp):
    mesh = plsc.VectorSubcoreMesh(core_axis_name="c", subcore_axis_name="s")
    out_type = jax.ShapeDtypeStruct((N, D_OUT), jnp.float32)
    scratch = [
        pltpu.VMEM((G_CHUNK,), jnp.int32),
        pltpu.VMEM((G_CHUNK, D_OUT), jnp.float32),
        pltpu.VMEM((G_CHUNK, D_OUT), jnp.float32),
        pltpu.SemaphoreType.DMA,
    ]
    kfn = pl.kernel(_gather_body, out_type=out_type, mesh=mesh,
                    scratch_types=scratch,
                    compiler_params=pltpu.CompilerParams(
                        needs_layout_passes=False,
                        use_tc_tiling_on_sc=False))
    return kfn(x1o, x2o, grp)


@jax.jit
def kernel(x1, edge_index1, x2, edge_index2, group_assignment,
           W1, att_src1, att_dst1, b1, W2, att_src2, att_dst2, b2):
    xp1, xp2, as1, ad1, m1, as2, ad2, m2 = _dense_call(
        x1, x2, W1, W2, att_src1, att_dst1, att_src2, att_dst2)

    padt = lambda a: jnp.pad(a.reshape(-1), (0, N_PAD - N))
    padr = lambda a: jnp.pad(a, ((0, N_PAD - N), (0, 0)))

    x1_out, x2_out = _edge_call(
        edge_index1, edge_index2, padt(as1), padt(ad1), padt(m1),
        padt(as2), padt(ad2), padt(m2), padr(xp1), padr(xp2), b1, b2)

    grp = group_assignment.astype(jnp.int32)
    x1_combined = _gather_call(x1_out, x2_out, grp)
    return (x1_combined, x2_out[:N])


# speedup vs baseline: 90.5743x; 1.1883x over previous
import jax
import jax.numpy as jnp
from jax import lax
from jax.experimental import pallas as pl
from jax.experimental.pallas import tpu as pltpu
from jax.experimental.pallas import tpu_sc as plsc

N = 10000
E = 320000
D_IN = 128
D_OUT = 16

NC = 2
NS = 16
L = 16

CHUNK = 128
K_CHUNKS = -(-E // (CHUNK * NS))
K_CHUNKS += K_CHUNKS % 2
E_TILE = K_CHUNKS * CHUNK
E_PAD = E_TILE * NS
N_PAD = 10240
ROWS_T = N_PAD // NS
G_TILE = N_PAD // (NC * NS)
G_CHUNK = 64
G_STEPS = G_TILE // G_CHUNK


def _leaky(x):
    return jnp.where(x >= 0, x, 0.2 * x)


def _row_dot(att, xp):
    return lax.dot_general(att, xp, (((1,), (1,)), ((), ())),
                           preferred_element_type=jnp.float32)


def _dense_body(x1, x2, w1, w2, s1, d1, s2, d2,
                xp1_o, xp2_o, as1_o, ad1_o, m1_o, as2_o, ad2_o, m2_o):
    zpad = jnp.zeros((N_PAD - N, D_OUT), jnp.float32)
    xp1 = jnp.concatenate(
        [jnp.dot(x1[...], w1[...], preferred_element_type=jnp.float32), zpad])
    xp2 = jnp.concatenate(
        [jnp.dot(x2[...], w2[...], preferred_element_type=jnp.float32), zpad])
    xp1_o[...] = xp1
    xp2_o[...] = xp2
    as1 = _row_dot(s1[...], xp1)
    ad1 = _row_dot(d1[...], xp1)
    as2 = _row_dot(s2[...], xp2)
    ad2 = _row_dot(d2[...], xp2)
    as1_o[...] = as1
    ad1_o[...] = ad1
    as2_o[...] = as2
    ad2_o[...] = ad2
    m1_o[...] = _leaky(jnp.max(as1) + ad1)
    m2_o[...] = _leaky(jnp.max(as2) + ad2)


def _dense_call(x1, x2, w1, w2, s1, d1, s2, d2):
    f32 = jnp.float32
    out_shape = [jax.ShapeDtypeStruct((N_PAD, D_OUT), f32)] * 2 + \
                [jax.ShapeDtypeStruct((1, N_PAD), f32)] * 6
    return pl.pallas_call(_dense_body, out_shape=out_shape)(
        x1, x2, w1, w2, s1.reshape(1, D_OUT), d1.reshape(1, D_OUT),
        s2.reshape(1, D_OUT), d2.reshape(1, D_OUT))


def _gat_on_core(s, ei_h, as_h, ad_h, m_h, xp_h, b_h, out_h,
                 as_t, ad_t, m_t, den_t, den16, idx2_b, e_b, rows_b, rbuf,
                 bias_v, acc, den_stage, sem_i, sem_g):
    iota16 = lax.iota(jnp.int32, L)

    pltpu.sync_copy(as_h.at[0], as_t)
    pltpu.sync_copy(ad_h.at[0], ad_t)
    pltpu.sync_copy(m_h.at[0], m_t)
    pltpu.sync_copy(b_h, bias_v)

    vbase = s * ROWS_T
    pltpu.sync_copy(xp_h.at[pl.ds(vbase, ROWS_T)], rbuf)

    def _zero_den(j, _):
        den_t[pl.ds(j * L, L)] = jnp.zeros((L,), jnp.float32)
        return 0
    lax.fori_loop(0, N_PAD // L, _zero_den, 0)

    def _self(jj, _):
        v0 = vbase + jj * L
        a = as_t[pl.ds(v0, L)] + ad_t[pl.ds(v0, L)]
        a = jnp.where(a >= 0, a, 0.2 * a)
        ev = jnp.exp(a - m_t[pl.ds(v0, L)])
        den_t[pl.ds(v0, L)] = ev
        for i in range(L):
            r = jj * L + i
            rbuf[r, :] = rbuf[r, :] * ev[i]
        return 0
    lax.fori_loop(0, ROWS_T // L, _self, 0)
    pltpu.sync_copy(rbuf, acc.at[pl.ds(vbase, ROWS_T)])
    plsc.subcore_barrier()

    base0 = s * E_TILE

    def phase(k, p, issue_next, prefetch_idx):
        if issue_next:
            pltpu.make_async_copy(ei_h.at[:, pl.ds(0, CHUNK)], idx2_b[1 - p],
                                  sem_i).wait()
            pltpu.async_copy(xp_h.at[idx2_b[1 - p].at[0]], rows_b[1 - p],
                             sem_g[1 - p])

        raw = base0 + k * CHUNK

        def _evec(j, _):
            si = idx2_b[p][0, pl.ds(j * L, L)]
            di = idx2_b[p][1, pl.ds(j * L, L)]
            a = plsc.load_gather(as_t, [si]) + plsc.load_gather(ad_t, [di])
            a = jnp.where(a >= 0, a, 0.2 * a)
            e = jnp.exp(a - plsc.load_gather(m_t, [di]))
            gid = jnp.minimum(raw, E - CHUNK) + j * L + iota16
            e = jnp.where(gid >= raw, e, 0.0)
            e_b[p][pl.ds(j * L, L)] = e
            plsc.addupdate_scatter(den_t, [di], e)
            return 0
        lax.fori_loop(0, CHUNK // L, _evec, 0, unroll=True)

        pltpu.make_async_copy(xp_h.at[pl.ds(0, CHUNK)], rows_b[p],
                              sem_g[p]).wait()

        def _scale(jj, _):
            w16 = e_b[p][pl.ds(jj * L, L)]
            for i in range(L):
                r = jj * L + i
                rows_b[p][r, :] = rows_b[p][r, :] * w16[i]
            return 0
        lax.fori_loop(0, CHUNK // L, _scale, 0)
        pltpu.sync_copy(rows_b[p], acc.at[idx2_b[p].at[1]], add=True)

        if prefetch_idx:
            b2 = jnp.minimum(raw + 2 * CHUNK, E - CHUNK)
            pltpu.async_copy(ei_h.at[:, pl.ds(b2, CHUNK)], idx2_b[p], sem_i)

    pltpu.sync_copy(ei_h.at[:, pl.ds(jnp.minimum(base0, E - CHUNK), CHUNK)],
                    idx2_b[0])
    pltpu.async_copy(xp_h.at[idx2_b[0].at[0]], rows_b[0], sem_g[0])
    pltpu.async_copy(
        ei_h.at[:, pl.ds(jnp.minimum(base0 + CHUNK, E - CHUNK), CHUNK)],
        idx2_b[1], sem_i)

    def _pair(k2, _):
        k = k2 * 2
        phase(k, 0, True, True)
        phase(k + 1, 1, True, True)
        return 0
    lax.fori_loop(0, K_CHUNKS // 2 - 1, _pair, 0)
    phase(K_CHUNKS - 2, 0, True, False)
    phase(K_CHUNKS - 1, 1, False, False)

    pltpu.sync_copy(den_t, den_stage.at[s])
    plsc.subcore_barrier()
    for t in range(NS):
        pltpu.sync_copy(den_stage.at[t, pl.ds(vbase, ROWS_T)], den16.at[t])

    pltpu.sync_copy(acc.at[pl.ds(vbase, ROWS_T)], rbuf)
    bias = bias_v[...]

    def _norm(jj, _):
        d = den16[0, pl.ds(jj * L, L)]
        for t in range(1, NS):
            d = d + den16[t, pl.ds(jj * L, L)]
        inv = 1.0 / d
        for i in range(L):
            r = jj * L + i
            rbuf[r, :] = rbuf[r, :] * inv[i] + bias
        return 0
    lax.fori_loop(0, ROWS_T // L, _norm, 0)
    pltpu.sync_copy(rbuf, out_h.at[pl.ds(vbase, ROWS_T)])


def _edge_body(ei1, ei2, as1, ad1, m1, as2, ad2, m2, xp1, xp2, b1, b2,
               x1o, x2o,
               as_t, ad_t, m_t, den_t, den16, idx2_0, idx2_1, e_c0, e_c1,
               rows0, rows1, rbuf, bias_v, acc, den_stage,
               sem_i, sem_g0, sem_g1):
    c = lax.axis_index("c")
    s = lax.axis_index("s")
    idx2_b = (idx2_0, idx2_1)
    e_b = (e_c0, e_c1)
    rows_b = (rows0, rows1)
    sem_g = (sem_g0, sem_g1)

    @pl.when(c == 0)
    def _():
        _gat_on_core(s, ei1, as1, ad1, m1, xp1, b1, x1o,
                     as_t, ad_t, m_t, den_t, den16, idx2_b, e_b, rows_b,
                     rbuf, bias_v, acc, den_stage, sem_i, sem_g)

    @pl.when(c == 1)
    def _():
        _gat_on_core(s, ei2, as2, ad2, m2, xp2, b2, x2o,
                     as_t, ad_t, m_t, den_t, den16, idx2_b, e_b, rows_b,
                     rbuf, bias_v, acc, den_stage, sem_i, sem_g)


def _edge_call(ei1, ei2, as1, ad1, m1, as2, ad2, m2, xp1, xp2, b1, b2):
    f32 = jnp.float32
    mesh = plsc.VectorSubcoreMesh(core_axis_name="c", subcore_axis_name="s")
    out_type = [
        jax.ShapeDtypeStruct((N_PAD, D_OUT), f32),
        jax.ShapeDtypeStruct((N_PAD, D_OUT), f32),
    ]
    scratch = [
        pltpu.VMEM((N_PAD,), f32),
        pltpu.VMEM((N_PAD,), f32),
        pltpu.VMEM((N_PAD,), f32),
        pltpu.VMEM((N_PAD,), f32),
        pltpu.VMEM((NS, ROWS_T), f32),
        pltpu.VMEM((2, CHUNK), jnp.int32),
        pltpu.VMEM((2, CHUNK), jnp.int32),
        pltpu.VMEM((CHUNK,), f32),
        pltpu.VMEM((CHUNK,), f32),
        pltpu.VMEM((CHUNK, D_OUT), f32),
        pltpu.VMEM((CHUNK, D_OUT), f32),
        pltpu.VMEM((ROWS_T, D_OUT), f32),
        pltpu.VMEM((L,), f32),
        pltpu.VMEM_SHARED((N_PAD, D_OUT), f32),
        pltpu.VMEM_SHARED((NS, N_PAD), f32),
        pltpu.SemaphoreType.DMA,
        pltpu.SemaphoreType.DMA,
        pltpu.SemaphoreType.DMA,
    ]
    kfn = pl.kernel(_edge_body, out_type=out_type, mesh=mesh,
                    scratch_types=scratch,
                    compiler_params=pltpu.CompilerParams(
                        needs_layout_passes=False,
                        use_tc_tiling_on_sc=False))
    return kfn(ei1, ei2, as1, ad1, m1, as2, ad2, m2, xp1, xp2, b1, b2)


def _gather_body(x1o, x2o, grp, out, idx_v, gbuf, xbuf, sem):
    c = lax.axis_index("c")
    s = lax.axis_index("s")
    base = (s * NC + c) * G_TILE

    def _step(k, _):
        off = jnp.minimum(base + k * G_CHUNK, N - G_CHUNK)
        pltpu.sync_copy(grp.at[pl.ds(off, G_CHUNK)], idx_v)
        cp = pltpu.async_copy(x2o.at[idx_v], gbuf, sem)
        pltpu.sync_copy(x1o.at[pl.ds(off, G_CHUNK)], xbuf)
        cp.wait()

        def _add(j, _):
            xbuf[j, :] = xbuf[j, :] + gbuf[j, :]
            return 0
        lax.fori_loop(0, G_CHUNK, _add, 0, unroll=8)
        pltpu.sync_copy(xbuf, out.at[pl.ds(off, G_CHUNK)])
        return 0
    lax.fori_loop(0, G_STEPS, _step, 0)


def _gather_call(x1o, x2o, grp):
    mesh = plsc.VectorSubcoreMesh(core_axis_name="c", subcore_axis_name="s")
    out_type = jax.ShapeDtypeStruct((N, D_OUT), jnp.float32)
    scratch = [
        pltpu.VMEM((G_CHUNK,), jnp.int32),
        pltpu.VMEM((G_CHUNK, D_OUT), jnp.float32),
        pltpu.VMEM((G_CHUNK, D_OUT), jnp.float32),
        pltpu.SemaphoreType.DMA,
    ]
    kfn = pl.kernel(_gather_body, out_type=out_type, mesh=mesh,
                    scratch_types=scratch,
                    compiler_params=pltpu.CompilerParams(
                        needs_layout_passes=False,
                        use_tc_tiling_on_sc=False))
    return kfn(x1o, x2o, grp)


@jax.jit
def kernel(x1, edge_index1, x2, edge_index2, group_assignment,
           W1, att_src1, att_dst1, b1, W2, att_src2, att_dst2, b2):
    xp1, xp2, as1, ad1, m1, as2, ad2, m2 = _dense_call(
        x1, x2, W1, W2, att_src1, att_dst1, att_src2, att_dst2)

    x1_out, x2_out = _edge_call(
        edge_index1, edge_index2, as1, ad1, m1, as2, ad2, m2, xp1, xp2,
        b1, b2)

    grp = group_assignment.astype(jnp.int32)
    x1_combined = _gather_call(x1_out, x2_out, grp)
    return (x1_combined, x2_out[:N])
